# baseline (device time: 275080 ns/iter reference)
import jax
import jax.numpy as jnp
from jax import lax
from jax.experimental import pallas as pl
from jax.experimental.pallas import tpu as pltpu

N_DEV = 16


def kernel(x, W1, W2):
    m, d = x.shape
    f = W1.shape[1]

    xb = x.astype(jnp.bfloat16)
    w1b = W1.astype(jnp.bfloat16)
    w2b = W2.astype(jnp.bfloat16)

    def body(x_ref, w1_ref, w2_ref, out_ref,
             xfull_ref, pstore_ref, rsc_ref, rs_sendbuf_ref,
             ag_send, ag_recv, rs_send, rs_recv):
        my = lax.axis_index("i")
        left = lax.rem(my - 1 + N_DEV, N_DEV)
        right = lax.rem(my + 1, N_DEV)

        barrier_sem = pltpu.get_barrier_semaphore()
        for nbr in (left, right):
            pl.semaphore_signal(
                barrier_sem, inc=1,
                device_id=(nbr,), device_id_type=pl.DeviceIdType.MESH,
            )
        pl.semaphore_wait(barrier_sem, 2)

        xfull_ref[pl.ds(my * m, m), :] = x_ref[...]
        for h in range(N_DEV - 1):
            o = lax.rem(my - h + N_DEV, N_DEV)
            src = x_ref.at[...] if h == 0 else xfull_ref.at[pl.ds(o * m, m), :]
            rdma = pltpu.make_async_remote_copy(
                src_ref=src,
                dst_ref=xfull_ref.at[pl.ds(o * m, m), :],
                send_sem=ag_send.at[h],
                recv_sem=ag_recv.at[h],
                device_id=(right,),
                device_id_type=pl.DeviceIdType.MESH,
            )
            rdma.start()
            rdma.wait()

        for j in range(N_DEV):
            xj = xfull_ref[pl.ds(j * m, m), :]
            h1 = jnp.dot(xj, w1_ref[...], preferred_element_type=jnp.float32)
            h1 = h1 * (1.0 / (1.0 + jnp.exp(-h1)))
            p = jnp.dot(h1.astype(jnp.bfloat16), w2_ref[...],
                        preferred_element_type=jnp.float32)
            pstore_ref[pl.ds(j * m, m), :] = p.astype(jnp.bfloat16)

        for s in range(1, N_DEV):
            b = lax.rem(my + s, N_DEV)
            if s == 1:
                src = pstore_ref.at[pl.ds(b * m, m), :]
            else:
                src = rs_sendbuf_ref.at[...]
            rdma = pltpu.make_async_remote_copy(
                src_ref=src,
                dst_ref=rsc_ref.at[s - 1],
                send_sem=rs_send.at[s - 1],
                recv_sem=rs_recv.at[s - 1],
                device_id=(left,),
                device_id_type=pl.DeviceIdType.MESH,
            )
            rdma.start()
            rdma.wait()
            r = lax.rem(my + s + 1, N_DEV)
            acc = (rsc_ref[s - 1].astype(jnp.float32)
                   + pstore_ref[pl.ds(r * m, m), :].astype(jnp.float32))
            if s < N_DEV - 1:
                rs_sendbuf_ref[...] = acc.astype(jnp.bfloat16)
            else:
                out_ref[...] = acc

    return pl.pallas_call(
        body,
        out_shape=jax.ShapeDtypeStruct((m, d), jnp.float32),
        in_specs=[
            pl.BlockSpec(memory_space=pltpu.VMEM),
            pl.BlockSpec(memory_space=pltpu.VMEM),
            pl.BlockSpec(memory_space=pltpu.VMEM),
        ],
        out_specs=pl.BlockSpec(memory_space=pltpu.VMEM),
        scratch_shapes=[
            pltpu.VMEM((N_DEV * m, d), jnp.bfloat16),
            pltpu.VMEM((N_DEV * m, d), jnp.bfloat16),
            pltpu.VMEM((N_DEV - 1, m, d), jnp.bfloat16),
            pltpu.VMEM((m, d), jnp.bfloat16),
            pltpu.SemaphoreType.DMA((N_DEV - 1,)),
            pltpu.SemaphoreType.DMA((N_DEV - 1,)),
            pltpu.SemaphoreType.DMA((N_DEV - 1,)),
            pltpu.SemaphoreType.DMA((N_DEV - 1,)),
        ],
        compiler_params=pltpu.CompilerParams(collective_id=0),
    )(xb, w1b, w2b)


# device time: 133737 ns/iter; 2.0569x vs baseline; 2.0569x over previous
import jax
import jax.numpy as jnp
from jax import lax
from jax.experimental import pallas as pl
from jax.experimental.pallas import tpu as pltpu

N_DEV = 16


def kernel(x, W1, W2):
    m, d = x.shape
    f = W1.shape[1]

    xb = x.astype(jnp.bfloat16)
    w1b = W1.astype(jnp.bfloat16)
    w2b = W2.astype(jnp.bfloat16)

    def body(x_ref, w1_ref, w2_ref, out_ref,
             xfull_ref, pstore_ref, rsr_buf, rsl_buf,
             agr_send, agr_recv, agl_send, agl_recv,
             rsr_send_sem, rsr_recv_sem, rsl_send_sem, rsl_recv_sem):
        my = lax.axis_index("i")
        left = lax.rem(my - 1 + N_DEV, N_DEV)
        right = lax.rem(my + 1, N_DEV)

        def idx(o):
            return lax.rem(o + 2 * N_DEV, N_DEV) * m

        def slot(o):
            return xfull_ref.at[pl.ds(idx(o), m), :]

        def pslot(b):
            return pstore_ref.at[pl.ds(idx(b), m), :]

        def compute_block(b):
            xj = xfull_ref[pl.ds(idx(b), m), :]
            h1 = jnp.dot(xj, w1_ref[...], preferred_element_type=jnp.float32)
            h1 = h1 * (1.0 / (1.0 + jnp.exp(-h1)))
            p = jnp.dot(h1.astype(jnp.bfloat16), w2_ref[...],
                        preferred_element_type=jnp.float32)
            pstore_ref[pl.ds(idx(b), m), :] = p.astype(jnp.bfloat16)

        sends = []

        def remote_copy(src, dst, ssem, rsem, tgt):
            r = pltpu.make_async_remote_copy(
                src_ref=src, dst_ref=dst, send_sem=ssem, recv_sem=rsem,
                device_id=(tgt,), device_id_type=pl.DeviceIdType.MESH)
            r.start()
            sends.append(r)
            return r

        barrier_sem = pltpu.get_barrier_semaphore()
        for nbr in (left, right):
            pl.semaphore_signal(
                barrier_sem, inc=1,
                device_id=(nbr,), device_id_type=pl.DeviceIdType.MESH)
        pl.semaphore_wait(barrier_sem, 2)

        xfull_ref[pl.ds(my * m, m), :] = x_ref[...]
        agr = [None] * 8
        agl = [None] * 7
        remote_copy(x_ref, slot(my), agr_send.at[0], agr_recv.at[0], right)
        remote_copy(x_ref, slot(my), agl_send.at[0], agl_recv.at[0], left)
        for t in range(1, 9):
            agr[t - 1] = pltpu.make_async_remote_copy(
                src_ref=x_ref, dst_ref=slot(my - t),
                send_sem=agr_send.at[0], recv_sem=agr_recv.at[t - 1],
                device_id=(right,), device_id_type=pl.DeviceIdType.MESH)
        for t in range(1, 8):
            agl[t - 1] = pltpu.make_async_remote_copy(
                src_ref=x_ref, dst_ref=slot(my + t),
                send_sem=agl_send.at[0], recv_sem=agl_recv.at[t - 1],
                device_id=(left,), device_id_type=pl.DeviceIdType.MESH)

        compute_block(my)

        for t in range(1, 9):
            agr[t - 1].wait_recv()
            if t < 8:
                remote_copy(slot(my - t), slot(my - t),
                            agr_send.at[t], agr_recv.at[t], right)
            if t <= 7:
                agl[t - 1].wait_recv()
                if t < 7:
                    remote_copy(slot(my + t), slot(my + t),
                                agl_send.at[t], agl_recv.at[t], left)
            compute_block(my - t)
            if t <= 7:
                compute_block(my + t)
            if t == 7:
                remote_copy(pslot(my + 7), rsr_buf.at[0],
                            rsr_send_sem.at[0], rsr_recv_sem.at[0], right)

        remote_copy(pslot(my - 8), rsl_buf.at[0],
                    rsl_send_sem.at[0], rsl_recv_sem.at[0], left)

        rsr = [None] * 7
        rsl = [None] * 8
        for j in range(7):
            rsr[j] = pltpu.make_async_remote_copy(
                src_ref=rsr_buf.at[0], dst_ref=rsr_buf.at[j],
                send_sem=rsr_send_sem.at[0], recv_sem=rsr_recv_sem.at[j],
                device_id=(right,), device_id_type=pl.DeviceIdType.MESH)
        for j in range(8):
            rsl[j] = pltpu.make_async_remote_copy(
                src_ref=rsl_buf.at[0], dst_ref=rsl_buf.at[j],
                send_sem=rsl_send_sem.at[0], recv_sem=rsl_recv_sem.at[j],
                device_id=(left,), device_id_type=pl.DeviceIdType.MESH)

        for j in range(1, 7):
            rsr[j - 1].wait_recv()
            rsr_buf[j - 1] = (rsr_buf[j - 1].astype(jnp.float32)
                              + pslot(my + 7 - j)[...].astype(jnp.float32)
                              ).astype(jnp.bfloat16)
            remote_copy(rsr_buf.at[j - 1], rsr_buf.at[j],
                        rsr_send_sem.at[j], rsr_recv_sem.at[j], right)
            rsl[j - 1].wait_recv()
            rsl_buf[j - 1] = (rsl_buf[j - 1].astype(jnp.float32)
                              + pslot(my + j - 8)[...].astype(jnp.float32)
                              ).astype(jnp.bfloat16)
            remote_copy(rsl_buf.at[j - 1], rsl_buf.at[j],
                        rsl_send_sem.at[j], rsl_recv_sem.at[j], left)
        rsl[6].wait_recv()
        rsl_buf[6] = (rsl_buf[6].astype(jnp.float32)
                      + pslot(my - 1)[...].astype(jnp.float32)
                      ).astype(jnp.bfloat16)
        remote_copy(rsl_buf.at[6], rsl_buf.at[7],
                    rsl_send_sem.at[7], rsl_recv_sem.at[7], left)

        rsr[6].wait_recv()
        rsl[7].wait_recv()
        out_ref[...] = (pslot(my)[...].astype(jnp.float32)
                        + rsr_buf[6].astype(jnp.float32)
                        + rsl_buf[7].astype(jnp.float32))

        for r in sends:
            r.wait_send()

    return pl.pallas_call(
        body,
        out_shape=jax.ShapeDtypeStruct((m, d), jnp.float32),
        in_specs=[
            pl.BlockSpec(memory_space=pltpu.VMEM),
            pl.BlockSpec(memory_space=pltpu.VMEM),
            pl.BlockSpec(memory_space=pltpu.VMEM),
        ],
        out_specs=pl.BlockSpec(memory_space=pltpu.VMEM),
        scratch_shapes=[
            pltpu.VMEM((N_DEV * m, d), jnp.bfloat16),
            pltpu.VMEM((N_DEV * m, d), jnp.bfloat16),
            pltpu.VMEM((7, m, d), jnp.bfloat16),
            pltpu.VMEM((8, m, d), jnp.bfloat16),
            pltpu.SemaphoreType.DMA((8,)),
            pltpu.SemaphoreType.DMA((8,)),
            pltpu.SemaphoreType.DMA((7,)),
            pltpu.SemaphoreType.DMA((7,)),
            pltpu.SemaphoreType.DMA((7,)),
            pltpu.SemaphoreType.DMA((7,)),
            pltpu.SemaphoreType.DMA((8,)),
            pltpu.SemaphoreType.DMA((8,)),
        ],
        compiler_params=pltpu.CompilerParams(collective_id=0),
    )(xb, w1b, w2b)


# device time: 129279 ns/iter; 2.1278x vs baseline; 1.0345x over previous
import jax
import jax.numpy as jnp
from jax import lax
from jax.experimental import pallas as pl
from jax.experimental.pallas import tpu as pltpu

N_DEV = 16
RING = [0, 1, 5, 9, 13, 14, 10, 6, 2, 3, 7, 11, 15, 12, 8, 4]


def kernel(x, W1, W2):
    m, d = x.shape
    f = W1.shape[1]

    xb = x.astype(jnp.bfloat16)
    w1b = W1.astype(jnp.bfloat16)
    w2b = W2.astype(jnp.bfloat16)

    def body(x_ref, w1_ref, w2_ref, out_ref,
             xfull_ref, pstore_ref, rsr_buf, rsl_buf,
             agr_send, agr_recv, agl_send, agl_recv,
             rsr_send_sem, rsr_recv_sem, rsl_send_sem, rsl_recv_sem):
        my = lax.axis_index("i")

        def table(vals, s):
            r = jnp.int32(0)
            for k, v in enumerate(vals):
                r = r + jnp.int32(v) * (s == k).astype(jnp.int32)
            return r

        RANK = [0] * N_DEV
        for rk, pos in enumerate(RING):
            RANK[pos] = rk
        my_rank = table(RANK, my)
        pR = [table([RING[(j + k) % N_DEV] for j in range(N_DEV)], my_rank)
              for k in range(N_DEV)]
        pL = [pR[(N_DEV - k) % N_DEV] for k in range(N_DEV)]
        right = pR[1]
        left = pL[1]

        def idx(p):
            return p * m

        def slot(p):
            return xfull_ref.at[pl.ds(idx(p), m), :]

        def pslot(p):
            return pstore_ref.at[pl.ds(idx(p), m), :]

        def compute_block(p):
            xj = xfull_ref[pl.ds(idx(p), m), :]
            h1 = jnp.dot(xj, w1_ref[...], preferred_element_type=jnp.float32)
            h1 = h1 * (1.0 / (1.0 + jnp.exp(-h1)))
            pp = jnp.dot(h1.astype(jnp.bfloat16), w2_ref[...],
                         preferred_element_type=jnp.float32)
            pstore_ref[pl.ds(idx(p), m), :] = pp.astype(jnp.bfloat16)

        sends = []

        def remote_copy(src, dst, ssem, rsem, tgt):
            r = pltpu.make_async_remote_copy(
                src_ref=src, dst_ref=dst, send_sem=ssem, recv_sem=rsem,
                device_id=(tgt,), device_id_type=pl.DeviceIdType.MESH)
            r.start()
            sends.append(r)
            return r

        barrier_sem = pltpu.get_barrier_semaphore()
        for nbr in (left, right):
            pl.semaphore_signal(
                barrier_sem, inc=1,
                device_id=(nbr,), device_id_type=pl.DeviceIdType.MESH)
        pl.semaphore_wait(barrier_sem, 2)

        xfull_ref[pl.ds(my * m, m), :] = x_ref[...]
        remote_copy(x_ref, slot(my), agr_send.at[0], agr_recv.at[0], right)
        remote_copy(x_ref, slot(my), agl_send.at[0], agl_recv.at[0], left)
        agr = [pltpu.make_async_remote_copy(
                   src_ref=x_ref, dst_ref=slot(pL[t]),
                   send_sem=agr_send.at[0], recv_sem=agr_recv.at[t - 1],
                   device_id=(right,), device_id_type=pl.DeviceIdType.MESH)
               for t in range(1, 9)]
        agl = [pltpu.make_async_remote_copy(
                   src_ref=x_ref, dst_ref=slot(pR[t]),
                   send_sem=agl_send.at[0], recv_sem=agl_recv.at[t - 1],
                   device_id=(left,), device_id_type=pl.DeviceIdType.MESH)
               for t in range(1, 8)]

        compute_block(my)

        for t in range(1, 9):
            agr[t - 1].wait_recv()
            if t < 8:
                remote_copy(slot(pL[t]), slot(pL[t]),
                            agr_send.at[t], agr_recv.at[t], right)
            if t <= 7:
                agl[t - 1].wait_recv()
                if t < 7:
                    remote_copy(slot(pR[t]), slot(pR[t]),
                                agl_send.at[t], agl_recv.at[t], left)
            compute_block(pL[t])
            if t <= 7:
                compute_block(pR[t])
            if t == 7:
                remote_copy(pslot(pR[7]), rsr_buf.at[pl.ds(0, m), :],
                            rsr_send_sem.at[0], rsr_recv_sem.at[0], right)

        remote_copy(pslot(pL[8]), rsl_buf.at[pl.ds(0, m), :],
                    rsl_send_sem.at[0], rsl_recv_sem.at[0], left)

        rsr = [pltpu.make_async_remote_copy(
                   src_ref=rsr_buf.at[pl.ds(0, m), :],
                   dst_ref=rsr_buf.at[pl.ds(j * m, m), :],
                   send_sem=rsr_send_sem.at[0], recv_sem=rsr_recv_sem.at[j],
                   device_id=(right,), device_id_type=pl.DeviceIdType.MESH)
               for j in range(7)]
        rsl = [pltpu.make_async_remote_copy(
                   src_ref=rsl_buf.at[pl.ds(0, m), :],
                   dst_ref=rsl_buf.at[pl.ds(j * m, m), :],
                   send_sem=rsl_send_sem.at[0], recv_sem=rsl_recv_sem.at[j],
                   device_id=(left,), device_id_type=pl.DeviceIdType.MESH)
               for j in range(8)]

        def rs_hop(bufs, descs, j, block_pos, send_sems, recv_sems, tgt):
            descs[j - 1].wait_recv()
            src_rows = pl.ds((j - 1) * m, m)
            dst_rows = pl.ds(j * m, m)
            bufs[src_rows, :] = (
                bufs[src_rows, :].astype(jnp.float32)
                + pstore_ref[pl.ds(idx(block_pos), m), :].astype(jnp.float32)
            ).astype(jnp.bfloat16)
            remote_copy(bufs.at[src_rows, :], bufs.at[dst_rows, :],
                        send_sems.at[j], recv_sems.at[j], tgt)

        for j in range(1, 7):
            rs_hop(rsr_buf, rsr, j, pR[7 - j], rsr_send_sem, rsr_recv_sem,
                   right)
            rs_hop(rsl_buf, rsl, j, pL[8 - j], rsl_send_sem, rsl_recv_sem,
                   left)
        rs_hop(rsl_buf, rsl, 7, pL[1], rsl_send_sem, rsl_recv_sem, left)

        rsr[6].wait_recv()
        rsl[7].wait_recv()
        out_ref[...] = (pstore_ref[pl.ds(idx(my), m), :].astype(jnp.float32)
                        + rsr_buf[pl.ds(6 * m, m), :].astype(jnp.float32)
                        + rsl_buf[pl.ds(7 * m, m), :].astype(jnp.float32))

        for r in sends:
            r.wait_send()

    return pl.pallas_call(
        body,
        out_shape=jax.ShapeDtypeStruct((m, d), jnp.float32),
        in_specs=[
            pl.BlockSpec(memory_space=pltpu.VMEM),
            pl.BlockSpec(memory_space=pltpu.VMEM),
            pl.BlockSpec(memory_space=pltpu.VMEM),
        ],
        out_specs=pl.BlockSpec(memory_space=pltpu.VMEM),
        scratch_shapes=[
            pltpu.VMEM((N_DEV * m, d), jnp.bfloat16),
            pltpu.VMEM((N_DEV * m, d), jnp.bfloat16),
            pltpu.VMEM((7 * m, d), jnp.bfloat16),
            pltpu.VMEM((8 * m, d), jnp.bfloat16),
            pltpu.SemaphoreType.DMA((8,)),
            pltpu.SemaphoreType.DMA((8,)),
            pltpu.SemaphoreType.DMA((7,)),
            pltpu.SemaphoreType.DMA((7,)),
            pltpu.SemaphoreType.DMA((7,)),
            pltpu.SemaphoreType.DMA((7,)),
            pltpu.SemaphoreType.DMA((8,)),
            pltpu.SemaphoreType.DMA((8,)),
        ],
        compiler_params=pltpu.CompilerParams(collective_id=0),
    )(xb, w1b, w2b)


# device time: 105332 ns/iter; 2.6116x vs baseline; 1.2273x over previous
import jax
import jax.numpy as jnp
from jax import lax
from jax.experimental import pallas as pl
from jax.experimental.pallas import tpu as pltpu

N_DEV = 16
RING = [0, 1, 5, 9, 13, 14, 10, 6, 2, 3, 7, 11, 15, 12, 8, 4]
S = 2


def kernel(x, W1, W2):
    m, d = x.shape
    f = W1.shape[1]
    hm = m // S

    xb = x.astype(jnp.bfloat16)
    w1b = W1.astype(jnp.bfloat16)
    w2b = W2.astype(jnp.bfloat16)

    def body(x_ref, w1_ref, w2_ref, out_ref,
             xfull_ref, pstore_ref, rsr_buf, rsl_buf,
             agr_send, agr_recv, agl_send, agl_recv,
             rsr_send_sem, rsr_recv_sem, rsl_send_sem, rsl_recv_sem):
        my = lax.axis_index("i")

        def table(vals, s):
            r = jnp.int32(0)
            for k, v in enumerate(vals):
                r = r + jnp.int32(v) * (s == k).astype(jnp.int32)
            return r

        RANK = [0] * N_DEV
        for rk, pos in enumerate(RING):
            RANK[pos] = rk
        my_rank = table(RANK, my)
        pR = [table([RING[(j + k) % N_DEV] for j in range(N_DEV)], my_rank)
              for k in range(N_DEV)]
        pL = [pR[(N_DEV - k) % N_DEV] for k in range(N_DEV)]
        right = pR[1]
        left = pL[1]

        def idx(p):
            return p * m

        def slot(p, q=None):
            if q is None:
                return xfull_ref.at[pl.ds(idx(p), m), :]
            return xfull_ref.at[pl.ds(idx(p) + q * hm, hm), :]

        def pslot(p, q=None):
            if q is None:
                return pstore_ref.at[pl.ds(idx(p), m), :]
            return pstore_ref.at[pl.ds(idx(p) + q * hm, hm), :]

        def compute_block(p):
            xj = xfull_ref[pl.ds(idx(p), m), :]
            h1 = jnp.dot(xj, w1_ref[...], preferred_element_type=jnp.float32)
            h1 = h1 * (1.0 / (1.0 + jnp.exp(-h1)))
            pp = jnp.dot(h1.astype(jnp.bfloat16), w2_ref[...],
                         preferred_element_type=jnp.float32)
            pstore_ref[pl.ds(idx(p), m), :] = pp.astype(jnp.bfloat16)

        sends = []

        def remote_copy(src, dst, ssem, rsem, tgt):
            r = pltpu.make_async_remote_copy(
                src_ref=src, dst_ref=dst, send_sem=ssem, recv_sem=rsem,
                device_id=(tgt,), device_id_type=pl.DeviceIdType.MESH)
            r.start()
            sends.append(r)
            return r

        barrier_sem = pltpu.get_barrier_semaphore()
        for nbr in (left, right):
            pl.semaphore_signal(
                barrier_sem, inc=1,
                device_id=(nbr,), device_id_type=pl.DeviceIdType.MESH)
        pl.semaphore_wait(barrier_sem, 2)

        xfull_ref[pl.ds(my * m, m), :] = x_ref[...]
        for q in range(S):
            remote_copy(x_ref.at[pl.ds(q * hm, hm), :], slot(my, q),
                        agr_send.at[q], agr_recv.at[q], right)
            remote_copy(x_ref.at[pl.ds(q * hm, hm), :], slot(my, q),
                        agl_send.at[q], agl_recv.at[q], left)
        agr = [[pltpu.make_async_remote_copy(
                    src_ref=x_ref.at[pl.ds(q * hm, hm), :],
                    dst_ref=slot(pL[t], q),
                    send_sem=agr_send.at[0],
                    recv_sem=agr_recv.at[(t - 1) * S + q],
                    device_id=(right,), device_id_type=pl.DeviceIdType.MESH)
                for q in range(S)] for t in range(1, 9)]
        agl = [[pltpu.make_async_remote_copy(
                    src_ref=x_ref.at[pl.ds(q * hm, hm), :],
                    dst_ref=slot(pR[t], q),
                    send_sem=agl_send.at[0],
                    recv_sem=agl_recv.at[(t - 1) * S + q],
                    device_id=(left,), device_id_type=pl.DeviceIdType.MESH)
                for q in range(S)] for t in range(1, 8)]

        compute_block(my)

        for t in range(1, 9):
            for q in range(S):
                agr[t - 1][q].wait_recv()
                if t < 8:
                    remote_copy(slot(pL[t], q), slot(pL[t], q),
                                agr_send.at[t * S + q],
                                agr_recv.at[t * S + q], right)
                if t <= 7:
                    agl[t - 1][q].wait_recv()
                    if t < 7:
                        remote_copy(slot(pR[t], q), slot(pR[t], q),
                                    agl_send.at[t * S + q],
                                    agl_recv.at[t * S + q], left)
            if t >= 7:
                compute_block(pL[t])
            if t <= 7:
                compute_block(pR[t])
            if t == 7:
                for q in range(S):
                    remote_copy(pslot(pR[7], q),
                                rsr_buf.at[pl.ds(q * hm, hm), :],
                                rsr_send_sem.at[q], rsr_recv_sem.at[q],
                                right)

        for q in range(S):
            remote_copy(pslot(pL[8], q), rsl_buf.at[pl.ds(q * hm, hm), :],
                        rsl_send_sem.at[q], rsl_recv_sem.at[q], left)

        rsr = [[pltpu.make_async_remote_copy(
                    src_ref=rsr_buf.at[pl.ds(q * hm, hm), :],
                    dst_ref=rsr_buf.at[pl.ds(j * m + q * hm, hm), :],
                    send_sem=rsr_send_sem.at[0],
                    recv_sem=rsr_recv_sem.at[j * S + q],
                    device_id=(right,), device_id_type=pl.DeviceIdType.MESH)
                for q in range(S)] for j in range(7)]
        rsl = [[pltpu.make_async_remote_copy(
                    src_ref=rsl_buf.at[pl.ds(q * hm, hm), :],
                    dst_ref=rsl_buf.at[pl.ds(j * m + q * hm, hm), :],
                    send_sem=rsl_send_sem.at[0],
                    recv_sem=rsl_recv_sem.at[j * S + q],
                    device_id=(left,), device_id_type=pl.DeviceIdType.MESH)
                for q in range(S)] for j in range(8)]

        def rs_part(bufs, descs, j, q, block_pos, send_sems, recv_sems, tgt):
            descs[j - 1][q].wait_recv()
            src_rows = pl.ds((j - 1) * m + q * hm, hm)
            dst_rows = pl.ds(j * m + q * hm, hm)
            bufs[src_rows, :] = (
                bufs[src_rows, :]
                + pstore_ref[pl.ds(idx(block_pos) + q * hm, hm), :])
            remote_copy(bufs.at[src_rows, :], bufs.at[dst_rows, :],
                        send_sems.at[j * S + q],
                        recv_sems.at[j * S + q], tgt)

        for j in range(1, 7):
            compute_block(pL[7 - j])
            for q in range(S):
                rs_part(rsr_buf, rsr, j, q, pR[7 - j],
                        rsr_send_sem, rsr_recv_sem, right)
                rs_part(rsl_buf, rsl, j, q, pL[8 - j],
                        rsl_send_sem, rsl_recv_sem, left)
        for q in range(S):
            rs_part(rsl_buf, rsl, 7, q, pL[1],
                    rsl_send_sem, rsl_recv_sem, left)

        for q in range(S):
            rsr[6][q].wait_recv()
            rsl[7][q].wait_recv()
        out_ref[...] = (pstore_ref[pl.ds(idx(my), m), :].astype(jnp.float32)
                        + rsr_buf[pl.ds(6 * m, m), :].astype(jnp.float32)
                        + rsl_buf[pl.ds(7 * m, m), :].astype(jnp.float32))

        for r in sends:
            r.wait_send()

    return pl.pallas_call(
        body,
        out_shape=jax.ShapeDtypeStruct((m, d), jnp.float32),
        in_specs=[
            pl.BlockSpec(memory_space=pltpu.VMEM),
            pl.BlockSpec(memory_space=pltpu.VMEM),
            pl.BlockSpec(memory_space=pltpu.VMEM),
        ],
        out_specs=pl.BlockSpec(memory_space=pltpu.VMEM),
        scratch_shapes=[
            pltpu.VMEM((N_DEV * m, d), jnp.bfloat16),
            pltpu.VMEM((N_DEV * m, d), jnp.bfloat16),
            pltpu.VMEM((7 * m, d), jnp.bfloat16),
            pltpu.VMEM((8 * m, d), jnp.bfloat16),
            pltpu.SemaphoreType.DMA((8 * S,)),
            pltpu.SemaphoreType.DMA((8 * S,)),
            pltpu.SemaphoreType.DMA((7 * S,)),
            pltpu.SemaphoreType.DMA((7 * S,)),
            pltpu.SemaphoreType.DMA((7 * S,)),
            pltpu.SemaphoreType.DMA((7 * S,)),
            pltpu.SemaphoreType.DMA((8 * S,)),
            pltpu.SemaphoreType.DMA((8 * S,)),
        ],
        compiler_params=pltpu.CompilerParams(collective_id=0),
    )(xb, w1b, w2b)
